# baseline (speedup 1.0000x reference)
import functools

import jax
import jax.numpy as jnp
from jax import lax
from jax.experimental import pallas as pl
from jax.experimental.pallas import tpu as pltpu
from jax.experimental.pallas import tpu_sc as plsc

NN = 10000
EE = 320000
DD = 128
HH = 64
NP = 10240

NC = 2
NS = 16
NWK = NC * NS
CH = 128
NCHUNK = 80
EPW = NCHUNK * CH
EPAD = NWK * EPW
RPT = NP // NS

BN = 512


def _make_spmm(width, k_slow, k_fast, slow_cid):
  kmax = max(k_slow, k_fast)
  mesh = plsc.VectorSubcoreMesh(core_axis_name="c", subcore_axis_name="s")

  @functools.partial(
      pl.kernel,
      out_type=jax.ShapeDtypeStruct((NC, NP, width), jnp.float32),
      mesh=mesh,
      compiler_params=pltpu.CompilerParams(use_tc_tiling_on_sc=False),
      scratch_types=[
          pltpu.VMEM((kmax, CH), jnp.int32),
          pltpu.VMEM((kmax, CH), jnp.int32),
          pltpu.VMEM((CH, width), jnp.float32),
          pltpu.VMEM((CH, width), jnp.float32),
          pltpu.VMEM((CH, width), jnp.float32),
          pltpu.VMEM_SHARED((NP, width), jnp.float32),
          pltpu.VMEM_SHARED((NP, width), jnp.float32),
          [pltpu.SemaphoreType.DMA] * 3,
          [pltpu.SemaphoreType.DMA] * 3,
      ],
  )
  def spmm(table, srcs, dsts, zeros, out,
           src_v, dst_v, buf0, buf1, buf2, acc, tab_s, gsem, ssem):
    bufs = (buf0, buf1, buf2)
    cid = lax.axis_index("c")
    sid = lax.axis_index("s")
    slow = cid == slow_cid
    k_here = lax.select(slow, k_slow, k_fast)
    base = lax.select(slow, sid * k_slow, NS * k_slow + sid * k_fast)
    pltpu.sync_copy(srcs.at[pl.ds(base, kmax)], src_v)
    pltpu.sync_copy(dsts.at[pl.ds(base, kmax)], dst_v)
    sl = pl.ds(sid * RPT, RPT)
    pltpu.sync_copy(table.at[sl], tab_s.at[sl])
    pltpu.sync_copy(zeros.at[sl], acc.at[sl])
    plsc.subcore_barrier()

    for b in range(3):
      pltpu.async_copy(tab_s.at[src_v.at[b]], bufs[b], gsem[b])

    @pl.loop(0, k_here, step=3)
    def _(j):
      for b in range(3):
        jj = j + b
        pltpu.make_async_copy(tab_s.at[src_v.at[jj]], bufs[b], gsem[b]).wait()
        pltpu.make_async_copy(
            bufs[b], acc.at[dst_v.at[jj]], ssem[b]).start(add=True)
      for b in range(3):
        jj3 = j + 3 + b

        @pl.when(jj3 < k_here)
        def _():
          pltpu.make_async_copy(
              bufs[b], acc.at[dst_v.at[jj3 - 3]], ssem[b]).wait()
          pltpu.async_copy(tab_s.at[src_v.at[jj3]], bufs[b], gsem[b])

    for b in range(3):
      pltpu.make_async_copy(
          bufs[b], acc.at[dst_v.at[k_here - 3 + b]], ssem[b]).wait()

    plsc.subcore_barrier()
    pltpu.sync_copy(acc.at[sl], out.at[cid, sl])

  return spmm


def _make_deg(k_slow, k_fast, slow_cid):
  kmax = max(k_slow, k_fast)
  mesh = plsc.VectorSubcoreMesh(core_axis_name="c", subcore_axis_name="s")

  @functools.partial(
      pl.kernel,
      out_type=jax.ShapeDtypeStruct((NC, NP, 8), jnp.float32),
      mesh=mesh,
      compiler_params=pltpu.CompilerParams(use_tc_tiling_on_sc=False),
      scratch_types=[
          pltpu.VMEM((kmax, CH), jnp.int32),
          pltpu.VMEM((CH, 8), jnp.float32),
          pltpu.VMEM_SHARED((NP, 8), jnp.float32),
          [pltpu.SemaphoreType.DMA] * 4,
      ],
  )
  def deg(ones, dsts, zeros, out, dst_v, ones_v, acc, ssem):
    cid = lax.axis_index("c")
    sid = lax.axis_index("s")
    slow = cid == slow_cid
    k_here = lax.select(slow, k_slow, k_fast)
    base = lax.select(slow, sid * k_slow, NS * k_slow + sid * k_fast)
    pltpu.sync_copy(dsts.at[pl.ds(base, kmax)], dst_v)
    pltpu.sync_copy(ones, ones_v)
    sl = pl.ds(sid * RPT, RPT)
    pltpu.sync_copy(zeros.at[sl], acc.at[sl])
    plsc.subcore_barrier()

    @pl.loop(0, k_here, step=4)
    def _(j):
      for b in range(4):
        jj = j + b

        @pl.when(jj >= 4)
        def _():
          pltpu.make_async_copy(
              ones_v, acc.at[dst_v.at[jj - 4]], ssem[b]).wait()

        pltpu.make_async_copy(
            ones_v, acc.at[dst_v.at[jj]], ssem[b]).start(add=True)

    for b in range(4):
      pltpu.make_async_copy(
          ones_v, acc.at[dst_v.at[k_here - 4 + b]], ssem[b]).wait()

    plsc.subcore_barrier()
    pltpu.sync_copy(acc.at[sl], out.at[cid, sl])

  return deg


FLATCH = 2592
SLOW = 0

_spmm64 = _make_spmm(HH, 81, 81, SLOW)
_spmm8 = _make_spmm(8, 81, 81, SLOW)
_deg = _make_deg(80, 80, SLOW)


def _mm1_body(x_ref, w_ref, o_ref):
  o_ref[...] = jnp.dot(x_ref[...], w_ref[...],
                       preferred_element_type=jnp.float32)


def _scale_body(deg_ref, h_ref, ht_ref, dinv_ref):
  deg = deg_ref[0, :, 0] + deg_ref[1, :, 0] + 1.0
  dinv = lax.rsqrt(deg)
  dinv_ref[...] = dinv[:, None]
  ht_ref[...] = h_ref[...] * dinv[:, None]


def _mid_body(a_ref, ht_ref, dinv_ref, b_ref, g_ref, be_ref, w_ref, o_ref):
  s = a_ref[0] + a_ref[1] + ht_ref[...]
  dinv = dinv_ref[...]
  u = s * dinv + b_ref[...]
  mu = jnp.mean(u, axis=1, keepdims=True)
  var = jnp.mean((u - mu) ** 2, axis=1, keepdims=True)
  un = (u - mu) * lax.rsqrt(var + 1e-5) * g_ref[...] + be_ref[...]
  r = jnp.maximum(un, 0.0)
  h = jnp.dot(r, w_ref[...], preferred_element_type=jnp.float32)
  o_ref[...] = h * dinv


def _fin_body(a_ref, ht_ref, dinv_ref, b3_ref, o_ref):
  s = a_ref[0, :, 0] + a_ref[1, :, 0] + ht_ref[:, 0]
  o_ref[...] = (s * dinv_ref[:, 0] + b3_ref[0])[:, None]


def _mm1(xp, W1):
  return pl.pallas_call(
      _mm1_body,
      grid=(NP // BN,),
      in_specs=[
          pl.BlockSpec((BN, DD), lambda i: (i, 0)),
          pl.BlockSpec((DD, HH), lambda i: (0, 0)),
      ],
      out_specs=pl.BlockSpec((BN, HH), lambda i: (i, 0)),
      out_shape=jax.ShapeDtypeStruct((NP, HH), jnp.float32),
  )(xp, W1)


def _scale(deg_out, h1):
  return pl.pallas_call(
      _scale_body,
      grid=(NP // BN,),
      in_specs=[
          pl.BlockSpec((NC, BN, 8), lambda i: (0, i, 0)),
          pl.BlockSpec((BN, HH), lambda i: (i, 0)),
      ],
      out_specs=[
          pl.BlockSpec((BN, HH), lambda i: (i, 0)),
          pl.BlockSpec((BN, 1), lambda i: (i, 0)),
      ],
      out_shape=[
          jax.ShapeDtypeStruct((NP, HH), jnp.float32),
          jax.ShapeDtypeStruct((NP, 1), jnp.float32),
      ],
  )(deg_out, h1)


def _mid(acc, ht, dinv, b, g, be, W, wout):
  return pl.pallas_call(
      _mid_body,
      grid=(NP // BN,),
      in_specs=[
          pl.BlockSpec((NC, BN, HH), lambda i: (0, i, 0)),
          pl.BlockSpec((BN, HH), lambda i: (i, 0)),
          pl.BlockSpec((BN, 1), lambda i: (i, 0)),
          pl.BlockSpec((1, HH), lambda i: (0, 0)),
          pl.BlockSpec((1, HH), lambda i: (0, 0)),
          pl.BlockSpec((1, HH), lambda i: (0, 0)),
          pl.BlockSpec((HH, wout), lambda i: (0, 0)),
      ],
      out_specs=pl.BlockSpec((BN, wout), lambda i: (i, 0)),
      out_shape=jax.ShapeDtypeStruct((NP, wout), jnp.float32),
  )(acc, ht, dinv, b.reshape(1, HH), g.reshape(1, HH), be.reshape(1, HH), W)


def _fin(acc, ht8, dinv, b3):
  return pl.pallas_call(
      _fin_body,
      grid=(NP // BN,),
      in_specs=[
          pl.BlockSpec((NC, BN, 8), lambda i: (0, i, 0)),
          pl.BlockSpec((BN, 8), lambda i: (i, 0)),
          pl.BlockSpec((BN, 1), lambda i: (i, 0)),
          pl.BlockSpec(memory_space=pltpu.SMEM),
      ],
      out_specs=pl.BlockSpec((BN, 1), lambda i: (i, 0)),
      out_shape=jax.ShapeDtypeStruct((NP, 1), jnp.float32),
  )(acc, ht8, dinv, b3)


def kernel(x, edge_index, W1, b1, g1, be1, W2, b2, g2, be2, W3, b3):
  src = edge_index[0]
  dst = edge_index[1]
  srcs = jnp.pad(src, (0, FLATCH * CH - EE)).reshape(FLATCH, CH)
  dsts = jnp.pad(dst, (0, FLATCH * CH - EE),
                 constant_values=NN).reshape(FLATCH, CH)
  xp = jnp.pad(x, ((0, NP - NN), (0, 0)))
  zeros64 = jnp.zeros((NP, HH), jnp.float32)
  zeros8 = jnp.zeros((NP, 8), jnp.float32)
  ones8 = jnp.ones((CH, 8), jnp.float32)
  W3p = jnp.tile(W3, (1, 8))

  deg_out = _deg(ones8, dsts, zeros8)
  h1 = _mm1(xp, W1)
  ht1, dinv = _scale(deg_out, h1)

  acc1 = _spmm64(ht1, srcs, dsts, zeros64)
  ht2 = _mid(acc1, ht1, dinv, b1, g1, be1, W2, HH)

  acc2 = _spmm64(ht2, srcs, dsts, zeros64)
  ht3 = _mid(acc2, ht2, dinv, b2, g2, be2, W3p, 8)

  acc3 = _spmm8(ht3, srcs, dsts, zeros8)
  out = _fin(acc3, ht3, dinv, b3)
  return out[:NN, 0]


# speedup vs baseline: 1.2087x; 1.2087x over previous
import functools

import jax
import jax.numpy as jnp
from jax import lax
from jax.experimental import pallas as pl
from jax.experimental.pallas import tpu as pltpu
from jax.experimental.pallas import tpu_sc as plsc

NN = 10000
EE = 320000
DD = 128
HH = 64
NP = 10240

NC = 2
NS = 16
NWK = NC * NS
CH = 128
NCHUNK = 80
EPW = NCHUNK * CH
EPAD = NWK * EPW
RPT = NP // NS

BN = 512


def _make_spmm(width, k_slow, k_fast, slow_cid):
  kmax = max(k_slow, k_fast)
  mesh = plsc.VectorSubcoreMesh(core_axis_name="c", subcore_axis_name="s")

  @functools.partial(
      pl.kernel,
      out_type=jax.ShapeDtypeStruct((NC, NP, width), jnp.float32),
      mesh=mesh,
      compiler_params=pltpu.CompilerParams(use_tc_tiling_on_sc=False),
      scratch_types=[
          pltpu.VMEM((kmax, CH), jnp.int32),
          pltpu.VMEM((kmax, CH), jnp.int32),
          pltpu.VMEM((CH, width), jnp.float32),
          pltpu.VMEM((CH, width), jnp.float32),
          pltpu.VMEM_SHARED((NP, width), jnp.float32),
          pltpu.VMEM_SHARED((NP, width), jnp.float32),
          [pltpu.SemaphoreType.DMA] * 2,
      ],
  )
  def spmm(table, srcs, dsts, zeros, out,
           src_v, dst_v, buf0, buf1, acc, tab_s, gsem):
    bufs = (buf0, buf1)
    cid = lax.axis_index("c")
    sid = lax.axis_index("s")
    slow = cid == slow_cid
    k_here = lax.select(slow, k_slow, k_fast)
    base = lax.select(slow, sid * k_slow, NS * k_slow + sid * k_fast)
    pltpu.sync_copy(srcs.at[pl.ds(base, kmax)], src_v)
    pltpu.sync_copy(dsts.at[pl.ds(base, kmax)], dst_v)
    sl = pl.ds(sid * RPT, RPT)
    pltpu.sync_copy(table.at[sl], tab_s.at[sl])
    pltpu.sync_copy(zeros.at[sl], acc.at[sl])
    plsc.subcore_barrier()

    pltpu.async_copy(tab_s.at[src_v.at[0]], bufs[0], gsem[0])

    @pl.loop(0, k_here, step=2)
    def _(j):
      pltpu.async_copy(tab_s.at[src_v.at[j + 1]], bufs[1], gsem[1])
      pltpu.make_async_copy(tab_s.at[src_v.at[j]], bufs[0], gsem[0]).wait()
      pltpu.sync_copy(bufs[0], acc.at[dst_v.at[j]], add=True)

      @pl.when(j + 2 < k_here)
      def _():
        pltpu.async_copy(tab_s.at[src_v.at[j + 2]], bufs[0], gsem[0])

      pltpu.make_async_copy(tab_s.at[src_v.at[j + 1]], bufs[1], gsem[1]).wait()
      pltpu.sync_copy(bufs[1], acc.at[dst_v.at[j + 1]], add=True)

    plsc.subcore_barrier()
    pltpu.sync_copy(acc.at[sl], out.at[cid, sl])

  return spmm


def _make_deg(k_slow, k_fast, slow_cid):
  kmax = max(k_slow, k_fast)
  mesh = plsc.VectorSubcoreMesh(core_axis_name="c", subcore_axis_name="s")

  @functools.partial(
      pl.kernel,
      out_type=jax.ShapeDtypeStruct((NC, NP, 8), jnp.float32),
      mesh=mesh,
      compiler_params=pltpu.CompilerParams(use_tc_tiling_on_sc=False),
      scratch_types=[
          pltpu.VMEM((kmax, CH), jnp.int32),
          pltpu.VMEM((CH, 8), jnp.float32),
          pltpu.VMEM_SHARED((NP, 8), jnp.float32),
          [pltpu.SemaphoreType.DMA] * 4,
      ],
  )
  def deg(ones, dsts, zeros, out, dst_v, ones_v, acc, ssem):
    cid = lax.axis_index("c")
    sid = lax.axis_index("s")
    slow = cid == slow_cid
    k_here = lax.select(slow, k_slow, k_fast)
    base = lax.select(slow, sid * k_slow, NS * k_slow + sid * k_fast)
    pltpu.sync_copy(dsts.at[pl.ds(base, kmax)], dst_v)
    pltpu.sync_copy(ones, ones_v)
    sl = pl.ds(sid * RPT, RPT)
    pltpu.sync_copy(zeros.at[sl], acc.at[sl])
    plsc.subcore_barrier()

    @pl.loop(0, k_here, step=4)
    def _(j):
      for b in range(4):
        jj = j + b

        @pl.when(jj >= 4)
        def _():
          pltpu.make_async_copy(
              ones_v, acc.at[dst_v.at[jj - 4]], ssem[b]).wait()

        pltpu.make_async_copy(
            ones_v, acc.at[dst_v.at[jj]], ssem[b]).start(add=True)

    for b in range(4):
      pltpu.make_async_copy(
          ones_v, acc.at[dst_v.at[k_here - 4 + b]], ssem[b]).wait()

    plsc.subcore_barrier()
    pltpu.sync_copy(acc.at[sl], out.at[cid, sl])

  return deg


FLATCH = 2592
SLOW = 0

_spmm64 = _make_spmm(HH, 80, 80, SLOW)
_spmm8 = _make_spmm(8, 80, 80, SLOW)
_deg = _make_deg(80, 80, SLOW)


def _dinv_of(deg_ref):
  deg = deg_ref[0, :, 0] + deg_ref[1, :, 0] + 1.0
  return lax.rsqrt(deg)[:, None]


def _mm1_body(x_ref, w_ref, o_ref):
  o_ref[...] = jnp.dot(x_ref[...], w_ref[...],
                       preferred_element_type=jnp.float32)


def _scale_body(deg_ref, h_ref, ht_ref):
  ht_ref[...] = h_ref[...] * _dinv_of(deg_ref)


def _mid_body(deg_ref, a_ref, ht_ref, b_ref, g_ref, be_ref, w_ref, o_ref):
  dinv = _dinv_of(deg_ref)
  s = a_ref[0] + a_ref[1] + ht_ref[...]
  u = s * dinv + b_ref[...]
  mu = jnp.mean(u, axis=1, keepdims=True)
  var = jnp.mean((u - mu) ** 2, axis=1, keepdims=True)
  un = (u - mu) * lax.rsqrt(var + 1e-5) * g_ref[...] + be_ref[...]
  r = jnp.maximum(un, 0.0)
  h = jnp.dot(r, w_ref[...], preferred_element_type=jnp.float32)
  o_ref[...] = h * dinv


def _fin_body(deg_ref, a_ref, ht_ref, b3_ref, o_ref):
  dinv = _dinv_of(deg_ref)
  s = a_ref[0, :, 0] + a_ref[1, :, 0] + ht_ref[:, 0]
  o_ref[...] = s[:, None] * dinv + b3_ref[0]


def _full(shape):
  return pl.BlockSpec(shape, lambda: tuple(0 for _ in shape))


def _mm1(xp, W1):
  return pl.pallas_call(
      _mm1_body,
      in_specs=[_full((NP, DD)), _full((DD, HH))],
      out_specs=_full((NP, HH)),
      out_shape=jax.ShapeDtypeStruct((NP, HH), jnp.float32),
  )(xp, W1)


def _scale(deg_out, h1):
  return pl.pallas_call(
      _scale_body,
      in_specs=[_full((NC, NP, 8)), _full((NP, HH))],
      out_specs=_full((NP, HH)),
      out_shape=jax.ShapeDtypeStruct((NP, HH), jnp.float32),
  )(deg_out, h1)


def _mid(deg_out, acc, ht, b, g, be, W, wout):
  return pl.pallas_call(
      _mid_body,
      in_specs=[
          _full((NC, NP, 8)),
          _full((NC, NP, HH)),
          _full((NP, HH)),
          _full((1, HH)),
          _full((1, HH)),
          _full((1, HH)),
          _full((HH, wout)),
      ],
      out_specs=_full((NP, wout)),
      out_shape=jax.ShapeDtypeStruct((NP, wout), jnp.float32),
  )(deg_out, acc, ht, b.reshape(1, HH), g.reshape(1, HH), be.reshape(1, HH), W)


def _fin(deg_out, acc, ht8, b3):
  return pl.pallas_call(
      _fin_body,
      in_specs=[
          _full((NC, NP, 8)),
          _full((NC, NP, 8)),
          _full((NP, 8)),
          pl.BlockSpec(memory_space=pltpu.SMEM),
      ],
      out_specs=_full((NP, 1)),
      out_shape=jax.ShapeDtypeStruct((NP, 1), jnp.float32),
  )(deg_out, acc, ht8, b3)


def kernel(x, edge_index, W1, b1, g1, be1, W2, b2, g2, be2, W3, b3):
  src = edge_index[0]
  dst = edge_index[1]
  srcs = jnp.pad(src, (0, FLATCH * CH - EE)).reshape(FLATCH, CH)
  dsts = jnp.pad(dst, (0, FLATCH * CH - EE),
                 constant_values=NN).reshape(FLATCH, CH)
  xp = jnp.pad(x, ((0, NP - NN), (0, 0)))
  zeros64 = jnp.zeros((NP, HH), jnp.float32)
  zeros8 = jnp.zeros((NP, 8), jnp.float32)
  ones8 = jnp.ones((CH, 8), jnp.float32)
  W3p = jnp.tile(W3, (1, 8))

  deg_out = _deg(ones8, dsts, zeros8)
  h1 = _mm1(xp, W1)
  ht1 = _scale(deg_out, h1)

  acc1 = _spmm64(ht1, srcs, dsts, zeros64)
  ht2 = _mid(deg_out, acc1, ht1, b1, g1, be1, W2, HH)

  acc2 = _spmm64(ht2, srcs, dsts, zeros64)
  ht3 = _mid(deg_out, acc2, ht2, b2, g2, be2, W3p, 8)

  acc3 = _spmm8(ht3, srcs, dsts, zeros8)
  out = _fin(deg_out, acc3, ht3, b3)
  return out[:NN, 0]


# speedup vs baseline: 1.2659x; 1.0474x over previous
import functools

import jax
import jax.numpy as jnp
from jax import lax
from jax.experimental import pallas as pl
from jax.experimental.pallas import tpu as pltpu
from jax.experimental.pallas import tpu_sc as plsc

NN = 10000
EE = 320000
DD = 128
HH = 64
NP = 10240

NC = 2
NS = 16
NWK = NC * NS
CH = 128
NCHUNK = 80
TPT = NN // NS
EPW = NCHUNK * CH
EPAD = NWK * EPW
RPT = NP // NS

BN = 512


def _make_spmm(width, k_slow, k_fast, slow_cid):
  kmax = max(k_slow, k_fast)
  mesh = plsc.VectorSubcoreMesh(core_axis_name="c", subcore_axis_name="s")

  @functools.partial(
      pl.kernel,
      out_type=jax.ShapeDtypeStruct((NC, NP, width), jnp.float32),
      mesh=mesh,
      compiler_params=pltpu.CompilerParams(use_tc_tiling_on_sc=False),
      scratch_types=[
          pltpu.VMEM((kmax, CH), jnp.int32),
          pltpu.VMEM((kmax, CH), jnp.int32),
          pltpu.VMEM((CH, width), jnp.float32),
          pltpu.VMEM((CH, width), jnp.float32),
          pltpu.VMEM_SHARED((NP, width), jnp.float32),
          pltpu.VMEM_SHARED((NN, width), jnp.float32),
          [pltpu.SemaphoreType.DMA] * 2,
      ],
  )
  def spmm(table, srcs, dsts, zeros, out,
           src_v, dst_v, buf0, buf1, acc, tab_s, gsem):
    bufs = (buf0, buf1)
    cid = lax.axis_index("c")
    sid = lax.axis_index("s")
    slow = cid == slow_cid
    k_here = lax.select(slow, k_slow, k_fast)
    base = lax.select(slow, sid * k_slow, NS * k_slow + sid * k_fast)
    pltpu.sync_copy(srcs.at[pl.ds(base, kmax)], src_v)
    pltpu.sync_copy(dsts.at[pl.ds(base, kmax)], dst_v)
    sl = pl.ds(sid * RPT, RPT)
    pltpu.sync_copy(table.at[pl.ds(sid * TPT, TPT)], tab_s.at[pl.ds(sid * TPT, TPT)])
    pltpu.sync_copy(zeros.at[sl], acc.at[sl])
    plsc.subcore_barrier()

    pltpu.async_copy(tab_s.at[src_v.at[0]], bufs[0], gsem[0])

    @pl.loop(0, k_here, step=2)
    def _(j):
      pltpu.async_copy(tab_s.at[src_v.at[j + 1]], bufs[1], gsem[1])
      pltpu.make_async_copy(tab_s.at[src_v.at[j]], bufs[0], gsem[0]).wait()
      pltpu.sync_copy(bufs[0], acc.at[dst_v.at[j]], add=True)

      @pl.when(j + 2 < k_here)
      def _():
        pltpu.async_copy(tab_s.at[src_v.at[j + 2]], bufs[0], gsem[0])

      pltpu.make_async_copy(tab_s.at[src_v.at[j + 1]], bufs[1], gsem[1]).wait()
      pltpu.sync_copy(bufs[1], acc.at[dst_v.at[j + 1]], add=True)

    plsc.subcore_barrier()
    pltpu.sync_copy(acc.at[sl], out.at[cid, sl])

  return spmm


def _make_deg(k_slow, k_fast, slow_cid):
  kmax = max(k_slow, k_fast)
  mesh = plsc.VectorSubcoreMesh(core_axis_name="c", subcore_axis_name="s")

  @functools.partial(
      pl.kernel,
      out_type=jax.ShapeDtypeStruct((NC, NP, 8), jnp.float32),
      mesh=mesh,
      compiler_params=pltpu.CompilerParams(use_tc_tiling_on_sc=False),
      scratch_types=[
          pltpu.VMEM((kmax, CH), jnp.int32),
          pltpu.VMEM((CH, 8), jnp.float32),
          pltpu.VMEM_SHARED((NP, 8), jnp.float32),
          [pltpu.SemaphoreType.DMA] * 4,
      ],
  )
  def deg(ones, dsts, zeros, out, dst_v, ones_v, acc, ssem):
    cid = lax.axis_index("c")
    sid = lax.axis_index("s")
    slow = cid == slow_cid
    k_here = lax.select(slow, k_slow, k_fast)
    base = lax.select(slow, sid * k_slow, NS * k_slow + sid * k_fast)
    pltpu.sync_copy(dsts.at[pl.ds(base, kmax)], dst_v)
    pltpu.sync_copy(ones, ones_v)
    sl = pl.ds(sid * RPT, RPT)
    pltpu.sync_copy(zeros.at[sl], acc.at[sl])
    plsc.subcore_barrier()

    @pl.loop(0, k_here, step=4)
    def _(j):
      for b in range(4):
        jj = j + b

        @pl.when(jj >= 4)
        def _():
          pltpu.make_async_copy(
              ones_v, acc.at[dst_v.at[jj - 4]], ssem[b]).wait()

        pltpu.make_async_copy(
            ones_v, acc.at[dst_v.at[jj]], ssem[b]).start(add=True)

    for b in range(4):
      pltpu.make_async_copy(
          ones_v, acc.at[dst_v.at[k_here - 4 + b]], ssem[b]).wait()

    plsc.subcore_barrier()
    pltpu.sync_copy(acc.at[sl], out.at[cid, sl])

  return deg


FLATCH = 2592
SLOW = 0

_spmm64 = _make_spmm(HH, 80, 80, SLOW)
_spmm8 = _make_spmm(8, 80, 80, SLOW)
_deg = _make_deg(80, 80, SLOW)


def _dinv_of(deg_ref):
  deg = deg_ref[0, :NN, 0] + deg_ref[1, :NN, 0] + 1.0
  return lax.rsqrt(deg)[:, None]


def _mm1_body(x_ref, w_ref, o_ref):
  o_ref[...] = jnp.dot(x_ref[...], w_ref[...],
                       preferred_element_type=jnp.float32)


def _scale_body(deg_ref, h_ref, ht_ref):
  ht_ref[...] = h_ref[...] * _dinv_of(deg_ref)


def _mid_body(deg_ref, a_ref, ht_ref, b_ref, g_ref, be_ref, w_ref, o_ref):
  dinv = _dinv_of(deg_ref)
  s = a_ref[0, :NN] + a_ref[1, :NN] + ht_ref[...]
  u = s * dinv + b_ref[...]
  mu = jnp.mean(u, axis=1, keepdims=True)
  var = jnp.mean((u - mu) ** 2, axis=1, keepdims=True)
  un = (u - mu) * lax.rsqrt(var + 1e-5) * g_ref[...] + be_ref[...]
  r = jnp.maximum(un, 0.0)
  h = jnp.dot(r, w_ref[...], preferred_element_type=jnp.float32)
  o_ref[...] = h * dinv


def _fin_body(deg_ref, a_ref, ht_ref, b3_ref, o_ref):
  dinv = _dinv_of(deg_ref)
  s = a_ref[0, :NN, 0] + a_ref[1, :NN, 0] + ht_ref[:, 0]
  o_ref[...] = s[:, None] * dinv + b3_ref[0]


def _prep_body(ei_ref, src_ref, dst_ref):
  src_ref[pl.ds(0, EE // CH), :] = ei_ref[0].reshape(EE // CH, CH)
  dst_ref[pl.ds(0, EE // CH), :] = ei_ref[1].reshape(EE // CH, CH)
  pad_rows = FLATCH - EE // CH
  src_ref[pl.ds(EE // CH, pad_rows), :] = jnp.zeros((pad_rows, CH), jnp.int32)
  dst_ref[pl.ds(EE // CH, pad_rows), :] = jnp.full((pad_rows, CH), NN,
                                                   jnp.int32)


def _full(shape):
  return pl.BlockSpec(shape, lambda: tuple(0 for _ in shape))


def _mm1(x, W1):
  return pl.pallas_call(
      _mm1_body,
      in_specs=[_full((NN, DD)), _full((DD, HH))],
      out_specs=_full((NN, HH)),
      out_shape=jax.ShapeDtypeStruct((NN, HH), jnp.float32),
  )(x, W1)


def _scale(deg_out, h1):
  return pl.pallas_call(
      _scale_body,
      in_specs=[_full((NC, NP, 8)), _full((NN, HH))],
      out_specs=_full((NN, HH)),
      out_shape=jax.ShapeDtypeStruct((NN, HH), jnp.float32),
  )(deg_out, h1)


def _mid(deg_out, acc, ht, b, g, be, W, wout):
  return pl.pallas_call(
      _mid_body,
      in_specs=[
          _full((NC, NP, 8)),
          _full((NC, NP, HH)),
          _full((NN, HH)),
          _full((1, HH)),
          _full((1, HH)),
          _full((1, HH)),
          _full((HH, wout)),
      ],
      out_specs=_full((NN, wout)),
      out_shape=jax.ShapeDtypeStruct((NN, wout), jnp.float32),
  )(deg_out, acc, ht, b.reshape(1, HH), g.reshape(1, HH), be.reshape(1, HH), W)


def _fin(deg_out, acc, ht8, b3):
  return pl.pallas_call(
      _fin_body,
      in_specs=[
          _full((NC, NP, 8)),
          _full((NC, NP, 8)),
          _full((NN, 8)),
          pl.BlockSpec(memory_space=pltpu.SMEM),
      ],
      out_specs=_full((NN, 1)),
      out_shape=jax.ShapeDtypeStruct((NN, 1), jnp.float32),
  )(deg_out, acc, ht8, b3)


def _prep(edge_index):
  return pl.pallas_call(
      _prep_body,
      in_specs=[_full((2, EE))],
      out_specs=[_full((FLATCH, CH)), _full((FLATCH, CH))],
      out_shape=[
          jax.ShapeDtypeStruct((FLATCH, CH), jnp.int32),
          jax.ShapeDtypeStruct((FLATCH, CH), jnp.int32),
      ],
  )(edge_index)


def kernel(x, edge_index, W1, b1, g1, be1, W2, b2, g2, be2, W3, b3):
  srcs, dsts = _prep(edge_index)
  zeros64 = jnp.zeros((NP, HH), jnp.float32)
  zeros8 = jnp.zeros((NP, 8), jnp.float32)
  ones8 = jnp.ones((CH, 8), jnp.float32)
  W3p = jnp.tile(W3, (1, 8))

  deg_out = _deg(ones8, dsts, zeros8)
  h1 = _mm1(x, W1)
  ht1 = _scale(deg_out, h1)

  acc1 = _spmm64(ht1, srcs, dsts, zeros64)
  ht2 = _mid(deg_out, acc1, ht1, b1, g1, be1, W2, HH)

  acc2 = _spmm64(ht2, srcs, dsts, zeros64)
  ht3 = _mid(deg_out, acc2, ht2, b2, g2, be2, W3p, 8)

  acc3 = _spmm8(ht3, srcs, dsts, zeros8)
  out = _fin(deg_out, acc3, ht3, b3)
  return out[:NN, 0]


# speedup vs baseline: 1.4967x; 1.1824x over previous
import functools

import jax
import jax.numpy as jnp
from jax import lax
from jax.experimental import pallas as pl
from jax.experimental.pallas import tpu as pltpu
from jax.experimental.pallas import tpu_sc as plsc

NN = 10000
EE = 320000
DD = 128
HH = 64
NP = 10240

NC = 2
NS = 16
NWK = NC * NS
CH = 128
NCHUNK = 80
TPT = NN // NS
EPW = NCHUNK * CH
EPAD = NWK * EPW
RPT = NP // NS

BN = 512


def _make_spmm(width, k_slow, k_fast, slow_cid):
  kmax = max(k_slow, k_fast)
  mesh = plsc.VectorSubcoreMesh(core_axis_name="c", subcore_axis_name="s")

  @functools.partial(
      pl.kernel,
      out_type=jax.ShapeDtypeStruct((NC, NP, width), jnp.float32),
      mesh=mesh,
      compiler_params=pltpu.CompilerParams(use_tc_tiling_on_sc=False),
      scratch_types=[
          pltpu.VMEM((kmax, CH), jnp.int32),
          pltpu.VMEM((kmax, CH), jnp.int32),
          pltpu.VMEM((CH, width), jnp.float32),
          pltpu.VMEM((CH, width), jnp.float32),
          pltpu.VMEM_SHARED((NP, width), jnp.float32),
          pltpu.VMEM_SHARED((NN, width), jnp.float32),
          [pltpu.SemaphoreType.DMA] * 2,
      ],
  )
  def spmm(table, srcs, dsts, zeros, out,
           src_v, dst_v, buf0, buf1, acc, tab_s, gsem):
    bufs = (buf0, buf1)
    cid = lax.axis_index("c")
    sid = lax.axis_index("s")
    slow = cid == slow_cid
    k_here = lax.select(slow, k_slow, k_fast)
    base = lax.select(slow, sid * k_slow, NS * k_slow + sid * k_fast)
    pltpu.sync_copy(srcs.at[pl.ds(base, kmax)], src_v)
    pltpu.sync_copy(dsts.at[pl.ds(base, kmax)], dst_v)
    sl = pl.ds(sid * RPT, RPT)
    pltpu.sync_copy(table.at[pl.ds(sid * TPT, TPT)], tab_s.at[pl.ds(sid * TPT, TPT)])
    pltpu.sync_copy(zeros.at[sl], acc.at[sl])
    plsc.subcore_barrier()

    pltpu.async_copy(tab_s.at[src_v.at[0]], bufs[0], gsem[0])

    @pl.loop(0, k_here, step=2)
    def _(j):
      pltpu.async_copy(tab_s.at[src_v.at[j + 1]], bufs[1], gsem[1])
      pltpu.make_async_copy(tab_s.at[src_v.at[j]], bufs[0], gsem[0]).wait()
      pltpu.sync_copy(bufs[0], acc.at[dst_v.at[j]], add=True)

      @pl.when(j + 2 < k_here)
      def _():
        pltpu.async_copy(tab_s.at[src_v.at[j + 2]], bufs[0], gsem[0])

      pltpu.make_async_copy(tab_s.at[src_v.at[j + 1]], bufs[1], gsem[1]).wait()
      pltpu.sync_copy(bufs[1], acc.at[dst_v.at[j + 1]], add=True)

    plsc.subcore_barrier()
    pltpu.sync_copy(acc.at[sl], out.at[cid, sl])

  return spmm


def _make_deg(k_slow, k_fast, slow_cid):
  kmax = max(k_slow, k_fast)
  mesh = plsc.VectorSubcoreMesh(core_axis_name="c", subcore_axis_name="s")

  @functools.partial(
      pl.kernel,
      out_type=jax.ShapeDtypeStruct((NC, NP, 8), jnp.float32),
      mesh=mesh,
      compiler_params=pltpu.CompilerParams(use_tc_tiling_on_sc=False),
      scratch_types=[
          pltpu.VMEM((kmax, CH), jnp.int32),
          pltpu.VMEM((CH, 8), jnp.float32),
          pltpu.VMEM_SHARED((NP, 8), jnp.float32),
          [pltpu.SemaphoreType.DMA] * 4,
      ],
  )
  def deg(ones, dsts, zeros, out, dst_v, ones_v, acc, ssem):
    cid = lax.axis_index("c")
    sid = lax.axis_index("s")
    slow = cid == slow_cid
    k_here = lax.select(slow, k_slow, k_fast)
    base = lax.select(slow, sid * k_slow, NS * k_slow + sid * k_fast)
    pltpu.sync_copy(dsts.at[pl.ds(base, kmax)], dst_v)
    pltpu.sync_copy(ones, ones_v)
    sl = pl.ds(sid * RPT, RPT)
    pltpu.sync_copy(zeros.at[sl], acc.at[sl])
    plsc.subcore_barrier()

    @pl.loop(0, k_here, step=4)
    def _(j):
      for b in range(4):
        jj = j + b

        @pl.when(jj >= 4)
        def _():
          pltpu.make_async_copy(
              ones_v, acc.at[dst_v.at[jj - 4]], ssem[b]).wait()

        pltpu.make_async_copy(
            ones_v, acc.at[dst_v.at[jj]], ssem[b]).start(add=True)

    for b in range(4):
      pltpu.make_async_copy(
          ones_v, acc.at[dst_v.at[k_here - 4 + b]], ssem[b]).wait()

    plsc.subcore_barrier()
    pltpu.sync_copy(acc.at[sl], out.at[cid, sl])

  return deg


FLATCH = 2592
SLOW = 0

_spmm64 = _make_spmm(HH, 80, 80, SLOW)
_spmm8 = _make_spmm(8, 80, 80, SLOW)
_deg = _make_deg(80, 80, SLOW)


PH = NN // 2


def _dinv_of(deg_ref):
  deg = deg_ref[0, :NN, 0] + deg_ref[1, :NN, 0] + 1.0
  return lax.rsqrt(deg)[:, None]


def _mm1_body(x_ref, w_ref, o_ref):
  o_ref[...] = jnp.dot(x_ref[...], w_ref[...],
                       preferred_element_type=jnp.float32)


def _scale_body(degp_ref, hp_ref, htp_ref, dp_ref, d16_ref):
  deg16 = degp_ref[0, :PH] + degp_ref[1, :PH] + 1.0
  d16 = lax.rsqrt(deg16)
  d16_ref[...] = d16
  pat = (lax.broadcasted_iota(jnp.int32, (16, 2 * HH), 1) // 8
         == lax.broadcasted_iota(jnp.int32, (16, 2 * HH), 0))
  dp = jnp.dot(d16, pat.astype(jnp.float32),
               preferred_element_type=jnp.float32)
  dp_ref[...] = dp
  htp_ref[...] = hp_ref[...] * dp


def _ln_half(u, eps=1e-5):
  mu = jnp.mean(u, axis=1, keepdims=True)
  var = jnp.mean((u - mu) ** 2, axis=1, keepdims=True)
  return (u - mu) * lax.rsqrt(var + eps)


def _mid_body(dp_ref, dsc_ref, a_ref, htp_ref, b_ref, g_ref, be_ref, w_ref,
              o_ref):
  dp = dp_ref[...]
  s = a_ref[0, :PH] + a_ref[1, :PH] + htp_ref[...]
  u = s * dp + b_ref[...]
  un = jnp.concatenate([_ln_half(u[:, :HH]), _ln_half(u[:, HH:])], axis=1)
  r = jnp.maximum(un * g_ref[...] + be_ref[...], 0.0)
  h = jnp.dot(r, w_ref[...], preferred_element_type=jnp.float32)
  o_ref[...] = h * dsc_ref[...]


def _fin_body(d16_ref, a_ref, htp_ref, b3_ref, o_ref):
  s = a_ref[0, :PH] + a_ref[1, :PH] + htp_ref[...]
  o_ref[...] = s * d16_ref[...] + b3_ref[0]


def _prep_body(ei_ref, src_ref, dst_ref):
  src_ref[pl.ds(0, EE // CH), :] = ei_ref[0].reshape(EE // CH, CH)
  dst_ref[pl.ds(0, EE // CH), :] = ei_ref[1].reshape(EE // CH, CH)
  pad_rows = FLATCH - EE // CH
  src_ref[pl.ds(EE // CH, pad_rows), :] = jnp.zeros((pad_rows, CH), jnp.int32)
  dst_ref[pl.ds(EE // CH, pad_rows), :] = jnp.full((pad_rows, CH), NN,
                                                   jnp.int32)


def _full(shape):
  return pl.BlockSpec(shape, lambda: tuple(0 for _ in shape))


def _mm1(xp, W1bd):
  return pl.pallas_call(
      _mm1_body,
      in_specs=[_full((PH, 2 * DD)), _full((2 * DD, 2 * HH))],
      out_specs=_full((PH, 2 * HH)),
      out_shape=jax.ShapeDtypeStruct((PH, 2 * HH), jnp.float32),
  )(xp, W1bd)


def _scale(deg_p, h1p):
  return pl.pallas_call(
      _scale_body,
      in_specs=[_full((NC, NP // 2, 16)), _full((PH, 2 * HH))],
      out_specs=[
          _full((PH, 2 * HH)),
          _full((PH, 2 * HH)),
          _full((PH, 16)),
      ],
      out_shape=[
          jax.ShapeDtypeStruct((PH, 2 * HH), jnp.float32),
          jax.ShapeDtypeStruct((PH, 2 * HH), jnp.float32),
          jax.ShapeDtypeStruct((PH, 16), jnp.float32),
      ],
  )(deg_p, h1p)


def _mid(dp, dsc, accp, htp, bp, gp, bep, Wbd, w2):
  return pl.pallas_call(
      _mid_body,
      in_specs=[
          _full((PH, 2 * HH)),
          _full((PH, w2)),
          _full((NC, NP // 2, 2 * HH)),
          _full((PH, 2 * HH)),
          _full((1, 2 * HH)),
          _full((1, 2 * HH)),
          _full((1, 2 * HH)),
          _full((2 * HH, w2)),
      ],
      out_specs=_full((PH, w2)),
      out_shape=jax.ShapeDtypeStruct((PH, w2), jnp.float32),
  )(dp, dsc, accp, htp, bp, gp, bep, Wbd)


def _fin(d16, accp, ht3p, b3):
  return pl.pallas_call(
      _fin_body,
      in_specs=[
          _full((PH, 16)),
          _full((NC, NP // 2, 16)),
          _full((PH, 16)),
          pl.BlockSpec(memory_space=pltpu.SMEM),
      ],
      out_specs=_full((PH, 16)),
      out_shape=jax.ShapeDtypeStruct((PH, 16), jnp.float32),
  )(d16, accp, ht3p, b3)


def _prep(edge_index):
  return pl.pallas_call(
      _prep_body,
      in_specs=[_full((2, EE))],
      out_specs=[_full((FLATCH, CH)), _full((FLATCH, CH))],
      out_shape=[
          jax.ShapeDtypeStruct((FLATCH, CH), jnp.int32),
          jax.ShapeDtypeStruct((FLATCH, CH), jnp.int32),
      ],
  )(edge_index)


def kernel(x, edge_index, W1, b1, g1, be1, W2, b2, g2, be2, W3, b3):
  srcs, dsts = _prep(edge_index)
  zeros64 = jnp.zeros((NP, HH), jnp.float32)
  zeros8 = jnp.zeros((NP, 8), jnp.float32)
  ones8 = jnp.ones((CH, 8), jnp.float32)
  zd = jnp.zeros((DD, HH), jnp.float32)
  zh = jnp.zeros((HH, HH), jnp.float32)
  z8 = jnp.zeros((HH, 8), jnp.float32)
  W1bd = jnp.block([[W1, zd], [zd, W1]])
  W2bd = jnp.block([[W2, zh], [zh, W2]])
  W3t = jnp.tile(W3, (1, 8))
  W3bd = jnp.block([[W3t, z8], [z8, W3t]])
  b1p, g1p, be1p = (jnp.tile(v, 2).reshape(1, 2 * HH) for v in (b1, g1, be1))
  b2p, g2p, be2p = (jnp.tile(v, 2).reshape(1, 2 * HH) for v in (b2, g2, be2))
  xp = x.reshape(PH, 2 * DD)

  deg_out = _deg(ones8, dsts, zeros8)
  h1p = _mm1(xp, W1bd)
  htp1, dp, d16 = _scale(deg_out.reshape(NC, NP // 2, 16), h1p)

  acc1 = _spmm64(htp1.reshape(NN, HH), srcs, dsts, zeros64)
  htp2 = _mid(dp, dp, acc1.reshape(NC, NP // 2, 2 * HH), htp1,
              b1p, g1p, be1p, W2bd, 2 * HH)

  acc2 = _spmm64(htp2.reshape(NN, HH), srcs, dsts, zeros64)
  ht3p = _mid(dp, d16, acc2.reshape(NC, NP // 2, 2 * HH), htp2,
              b2p, g2p, be2p, W3bd, 16)

  ht3 = ht3p.reshape(NN, 8)
  acc3 = _spmm8(ht3, srcs, dsts, zeros8)
  outp = _fin(d16, acc3.reshape(NC, NP // 2, 16), ht3p, b3)
  return outp.reshape(NN, 8)[:, 0]
